# baseline (speedup 1.0000x reference)
import functools

import jax
import jax.numpy as jnp
from jax import lax
from jax.experimental import pallas as pl
from jax.experimental.pallas import tpu as pltpu
from jax.experimental.pallas import tpu_sc as plsc

EMBED = 64
SCALE = 8.0
NW = 32
LANES = 16
BBLK = 128


@functools.lru_cache(maxsize=None)
def _build(BATCH, HIST, V):
    assert BATCH == NW * BBLK
    NBH = BATCH // BBLK
    EHI = EMBED // 8

    mesh = plsc.VectorSubcoreMesh(core_axis_name="c", subcore_axis_name="s")

    @functools.partial(
        pl.kernel,
        mesh=mesh,
        out_type=jax.ShapeDtypeStruct((HIST, EHI, NBH, 8, BBLK), jnp.float32),
        scratch_types=[
            pltpu.VMEM((HIST, BBLK), jnp.int32),
            pltpu.VMEM((BBLK, EMBED), jnp.float32),
            pltpu.VMEM((BBLK, EMBED), jnp.float32),
            pltpu.VMEM((EMBED, BBLK), jnp.float32),
            pltpu.VMEM((EMBED, BBLK), jnp.float32),
            pltpu.SemaphoreType.DMA,
            pltpu.SemaphoreType.DMA,
        ],
        compiler_params=pltpu.CompilerParams(use_tc_tiling_on_sc=False,
                                             needs_layout_passes=False),
    )
    def k(xt_hbm, lut_hbm, out_hbm, idx_v, ra, rb, ta, tb, gsem, osem):
        wid = lax.axis_index("s") * 2 + lax.axis_index("c")
        pltpu.sync_copy(xt_hbm.at[:, pl.ds(wid * BBLK, BBLK)], idx_v)
        iota = lax.iota(jnp.int32, LANES)

        def fire_gather(h, rbuf):
            pltpu.async_copy(lut_hbm.at[idx_v.at[h]], rbuf, gsem)

        def wait_gather(rbuf):
            pltpu.make_async_copy(lut_hbm.at[pl.ds(0, BBLK)], rbuf, gsem).wait()

        def drain_outs(tbuf):
            pltpu.make_async_copy(lut_hbm.at[pl.ds(0, BBLK)], tbuf, osem).wait()

        def transpose_store(h, rbuf, tbuf):
            def col(e, carry):
                for g in range(BBLK // LANES):
                    v = plsc.load_gather(rbuf, [iota + g * LANES,
                                                jnp.full((LANES,), e, jnp.int32)])
                    tbuf[e, pl.ds(g * LANES, LANES)] = v * SCALE
                return carry

            lax.fori_loop(0, EMBED, col, 0)
            for i in range(EHI):
                pltpu.async_copy(tbuf.at[pl.ds(8 * i, 8)],
                                 out_hbm.at[h, i, wid], osem)

        fire_gather(0, ra)

        def pair_body(p, carry):
            u0 = 2 * p
            wait_gather(ra)
            fire_gather(u0 + 1, rb)

            @pl.when(p >= 1)
            def _():
                drain_outs(ta)

            transpose_store(u0, ra, ta)

            @pl.when(p + 1 < HIST // 2)
            def _():
                fire_gather(u0 + 2, ra)

            @pl.when(p >= 1)
            def _():
                drain_outs(tb)

            wait_gather(rb)
            transpose_store(u0 + 1, rb, tb)
            return carry

        lax.fori_loop(0, HIST // 2, pair_body, 0)
        drain_outs(ta)
        drain_outs(tb)

    return k


def kernel(x, lut):
    BATCH, HIST = x.shape
    xt = jnp.transpose(x).astype(jnp.int32)
    out5 = _build(BATCH, HIST, lut.shape[0])(xt, lut)
    return (out5.transpose(2, 4, 0, 1, 3)
            .reshape(BATCH, HIST, EMBED))


# speedup vs baseline: 1.9376x; 1.9376x over previous
import functools

import jax
import jax.numpy as jnp
import numpy as _NP
from jax import lax
from jax.experimental import pallas as pl
from jax.experimental.pallas import tpu as pltpu
from jax.experimental.pallas import tpu_sc as plsc

EMBED = 64
SCALE = 8.0
NW = 32
LANES = 16
BBLK = 128


@functools.lru_cache(maxsize=None)
def _build(BATCH, HIST, V):
    assert BATCH == NW * BBLK
    NBH = BATCH // BBLK
    EHI = EMBED // 8

    mesh = plsc.VectorSubcoreMesh(core_axis_name="c", subcore_axis_name="s")

    @functools.partial(
        pl.kernel,
        mesh=mesh,
        out_type=jax.ShapeDtypeStruct((HIST, EHI, NBH, 8, BBLK), jnp.float32),
        scratch_types=[
            pltpu.VMEM((HIST, BBLK), jnp.int32),
            pltpu.VMEM((BBLK, EMBED), jnp.float32),
            pltpu.VMEM((BBLK, EMBED), jnp.float32),
            pltpu.VMEM((EMBED, BBLK), jnp.float32),
            pltpu.VMEM((EMBED, BBLK), jnp.float32),
            pltpu.SemaphoreType.DMA,
            pltpu.SemaphoreType.DMA,
        ],
        compiler_params=pltpu.CompilerParams(use_tc_tiling_on_sc=False,
                                             needs_layout_passes=False),
    )
    def k(xt_hbm, lut_hbm, out_hbm, idx_v, ra, rb, ta, tb, gsem, osem):
        wid = lax.axis_index("s") * 2 + lax.axis_index("c")
        pltpu.sync_copy(xt_hbm.at[:, pl.ds(wid * BBLK, BBLK)], idx_v)
        iota = lax.iota(jnp.int32, LANES)
        r_cs = [iota + LANES * g for g in range(BBLK // LANES)]

        def fire_gather(h, rbuf):
            pltpu.async_copy(lut_hbm.at[idx_v.at[h]], rbuf, gsem)

        def wait_gather(rbuf):
            pltpu.make_async_copy(lut_hbm.at[pl.ds(0, BBLK)], rbuf, gsem).wait()

        def drain_outs(tbuf):
            pltpu.make_async_copy(lut_hbm.at[pl.ds(0, BBLK)], tbuf, osem).wait()

        def transpose_store(h, rbuf, tbuf):
            for q in range(EMBED // LANES):
                def dbody(d, carry):
                    e_c = ((iota + d) & (LANES - 1)) + (LANES * q)
                    for g in range(BBLK // LANES):
                        v = plsc.load_gather(rbuf, [r_cs[g], e_c])
                        plsc.store_scatter(tbuf, [e_c, r_cs[g]], v * SCALE)
                    return carry

                lax.fori_loop(0, LANES, dbody, 0)
            for i in range(EHI):
                pltpu.async_copy(tbuf.at[pl.ds(8 * i, 8)],
                                 out_hbm.at[h, i, wid], osem)

        fire_gather(0, ra)

        def pair_body(p, carry):
            u0 = 2 * p
            wait_gather(ra)
            fire_gather(u0 + 1, rb)

            @pl.when(p >= 1)
            def _():
                drain_outs(ta)

            transpose_store(u0, ra, ta)

            @pl.when(p + 1 < HIST // 2)
            def _():
                fire_gather(u0 + 2, ra)

            @pl.when(p >= 1)
            def _():
                drain_outs(tb)

            wait_gather(rb)
            transpose_store(u0 + 1, rb, tb)
            return carry

        lax.fori_loop(0, HIST // 2, pair_body, 0)
        drain_outs(ta)
        drain_outs(tb)

    return k


def kernel(x, lut):
    BATCH, HIST = x.shape
    xt = jnp.transpose(x).astype(jnp.int32)
    out5 = _build(BATCH, HIST, lut.shape[0])(xt, lut)
    return (out5.transpose(2, 4, 0, 1, 3)
            .reshape(BATCH, HIST, EMBED))


# speedup vs baseline: 3.2092x; 1.6563x over previous
import functools

import jax
import jax.numpy as jnp
import numpy as _NP
from jax import lax
from jax.experimental import pallas as pl
from jax.experimental.pallas import tpu as pltpu
from jax.experimental.pallas import tpu_sc as plsc

EMBED = 64
SCALE = 8.0
NW = 32
LANES = 16
BBLK = 128


@functools.lru_cache(maxsize=None)
def _build(BATCH, HIST, V):
    assert BATCH == NW * BBLK
    NBH = BATCH // BBLK
    EHI = EMBED // 8

    mesh = plsc.VectorSubcoreMesh(core_axis_name="c", subcore_axis_name="s")

    @functools.partial(
        pl.kernel,
        mesh=mesh,
        out_type=jax.ShapeDtypeStruct((HIST, EHI, NBH, 8, BBLK), jnp.float32),
        scratch_types=[
            pltpu.VMEM((HIST, BBLK), jnp.int32),
            pltpu.VMEM((BBLK, EMBED), jnp.float32),
            pltpu.VMEM((BBLK, EMBED), jnp.float32),
            pltpu.VMEM((EMBED, BBLK), jnp.float32),
            pltpu.VMEM((EMBED, BBLK), jnp.float32),
            pltpu.SemaphoreType.DMA,
            pltpu.SemaphoreType.DMA,
        ],
        compiler_params=pltpu.CompilerParams(use_tc_tiling_on_sc=False,
                                             needs_layout_passes=False),
    )
    def k(xt_hbm, lut_hbm, out_hbm, idx_v, ra, rb, ta, tb, gsem, osem):
        wid = lax.axis_index("s") * 2 + lax.axis_index("c")
        pltpu.sync_copy(xt_hbm.at[:, pl.ds(wid * BBLK, BBLK)], idx_v)
        iota = lax.iota(jnp.int32, LANES)
        r_cs = [iota + LANES * g for g in range(BBLK // LANES)]

        def fire_gather(h, rbuf):
            pltpu.async_copy(lut_hbm.at[idx_v.at[h]], rbuf, gsem)

        def wait_gather(rbuf):
            pltpu.make_async_copy(lut_hbm.at[pl.ds(0, BBLK)], rbuf, gsem).wait()

        def drain_outs(tbuf):
            pltpu.make_async_copy(lut_hbm.at[pl.ds(0, BBLK)], tbuf, osem).wait()

        def transpose_store(h, rbuf, tbuf):
            for q in range(EMBED // LANES):
                @plsc.parallel_loop(0, LANES, unroll=2)
                def _(d, _q=q):
                    e_c = ((iota + d) & (LANES - 1)) + (LANES * _q)
                    for g in range(BBLK // LANES):
                        v = plsc.load_gather(rbuf, [r_cs[g], e_c])
                        plsc.store_scatter(tbuf, [e_c, r_cs[g]], v * SCALE)
            for i in range(EHI):
                pltpu.async_copy(tbuf.at[pl.ds(8 * i, 8)],
                                 out_hbm.at[h, i, wid], osem)

        fire_gather(0, ra)

        def pair_body(p, carry):
            u0 = 2 * p
            wait_gather(ra)
            fire_gather(u0 + 1, rb)

            @pl.when(p >= 1)
            def _():
                drain_outs(ta)

            transpose_store(u0, ra, ta)

            @pl.when(p + 1 < HIST // 2)
            def _():
                fire_gather(u0 + 2, ra)

            @pl.when(p >= 1)
            def _():
                drain_outs(tb)

            wait_gather(rb)
            transpose_store(u0 + 1, rb, tb)
            return carry

        lax.fori_loop(0, HIST // 2, pair_body, 0)
        drain_outs(ta)
        drain_outs(tb)

    return k


def kernel(x, lut):
    BATCH, HIST = x.shape
    xt = jnp.transpose(x).astype(jnp.int32)
    out5 = _build(BATCH, HIST, lut.shape[0])(xt, lut)
    return (out5.transpose(2, 4, 0, 1, 3)
            .reshape(BATCH, HIST, EMBED))
